# Initial kernel scaffold; baseline (speedup 1.0000x reference)
import jax
import jax.numpy as jnp
from jax.experimental import pallas as pl


def kernel(params, idx, targets):
    raise NotImplementedError("write your pallas kernel here")


# speedup vs baseline: 1.3155x; 1.3155x over previous
import functools

import jax
import jax.numpy as jnp
from jax import lax
from jax.experimental import pallas as pl
from jax.experimental.pallas import tpu as pltpu
from jax.experimental.pallas import tpu_sc as plsc

VOCAB = 1000
BT = 51200
NW = 32
RPW = BT // NW
CH = 32
NCH = RPW // CH
LANES = 16


def _lse_body(p_ref, o_ref):
    x = p_ref[...]
    m = jnp.max(x, axis=1)
    s = jnp.sum(jnp.exp(x - m[:, None]), axis=1)
    o_ref[...] = m + jnp.log(s)


def _row_lse(params):
    return pl.pallas_call(
        _lse_body,
        out_shape=jax.ShapeDtypeStruct((VOCAB,), jnp.float32),
    )(params)


_mesh = plsc.VectorSubcoreMesh(core_axis_name="c", subcore_axis_name="s")


@functools.partial(
    pl.kernel,
    mesh=_mesh,
    compiler_params=pltpu.CompilerParams(needs_layout_passes=False,
                                         use_tc_tiling_on_sc=False),
    out_type=(
        jax.ShapeDtypeStruct((BT, VOCAB), jnp.float32),
        jax.ShapeDtypeStruct((NW, LANES), jnp.float32),
    ),
    scratch_types=[
        pltpu.VMEM((NCH, CH), jnp.int32),
        pltpu.VMEM((NCH, CH), jnp.int32),
        pltpu.VMEM((VOCAB,), jnp.float32),
        pltpu.VMEM((CH, VOCAB), jnp.float32),
        pltpu.VMEM((LANES,), jnp.float32),
        pltpu.SemaphoreType.DMA,
    ],
)
def _sc_kernel(params_hbm, idx_hbm, tgt_hbm, lse_hbm, out_hbm, part_hbm,
               idx_v, tgt_v, lse_v, rows_v, acc_v, sem):
    wid = lax.axis_index("s") * 2 + lax.axis_index("c")
    base = wid * RPW
    pltpu.sync_copy(idx_hbm.at[wid], idx_v)
    pltpu.sync_copy(tgt_hbm.at[wid], tgt_v)
    pltpu.sync_copy(lse_hbm, lse_v)

    def chunk_body(c, acc):
        pltpu.async_copy(params_hbm.at[idx_v.at[c]], rows_v, sem).wait()
        for j in range(CH // LANES):
            i_vec = idx_v[c, pl.ds(j * LANES, LANES)]
            t_vec = tgt_v[c, pl.ds(j * LANES, LANES)]
            lse_vals = plsc.load_gather(lse_v, [i_vec])
            rid = lax.iota(jnp.int32, LANES) + j * LANES
            p_vals = plsc.load_gather(rows_v, [rid, t_vec])
            acc = acc + (lse_vals - p_vals)
        pltpu.sync_copy(rows_v, out_hbm.at[pl.ds(base + c * CH, CH)])
        return acc

    acc = lax.fori_loop(0, NCH, chunk_body,
                        jnp.zeros((LANES,), jnp.float32))
    acc_v[...] = acc
    pltpu.sync_copy(acc_v, part_hbm.at[wid])


def kernel(params, idx, targets):
    idx3 = idx.astype(jnp.int32).reshape(NW, NCH, CH)
    tgt3 = targets.astype(jnp.int32).reshape(NW, NCH, CH)
    lse = _row_lse(params)
    logits, partials = _sc_kernel(params, idx3, tgt3, lse)
    loss = jnp.sum(partials) / BT
    return (loss, logits)


# speedup vs baseline: 1.5610x; 1.1867x over previous
import functools

import jax
import jax.numpy as jnp
from jax import lax
from jax.experimental import pallas as pl
from jax.experimental.pallas import tpu as pltpu
from jax.experimental.pallas import tpu_sc as plsc

VOCAB = 1000
VPAD = 1024
NP = 8
PW = 128
LASTW = VOCAB - (NP - 1) * PW
BT = 51200
NW = 32
RPW = BT // NW
T = 80
NCH = RPW // T
LANES = 16


def _lse_body(p_ref, o_ref):
    x = p_ref[...]
    m = jnp.max(x, axis=1)
    s = jnp.sum(jnp.exp(x - m[:, None]), axis=1)
    lse = m + jnp.log(s)
    o_ref[...] = jnp.concatenate(
        [lse, jnp.zeros((VPAD - VOCAB,), jnp.float32)])


def _row_lse(params):
    return pl.pallas_call(
        _lse_body,
        out_shape=jax.ShapeDtypeStruct((VPAD,), jnp.float32),
    )(params)


_mesh = plsc.VectorSubcoreMesh(core_axis_name="c", subcore_axis_name="s")


@functools.partial(
    pl.kernel,
    mesh=_mesh,
    compiler_params=pltpu.CompilerParams(needs_layout_passes=False),
    out_type=(
        jax.ShapeDtypeStruct((BT, VOCAB), jnp.float32),
        jax.ShapeDtypeStruct((BT, PW), jnp.float32),
        jax.ShapeDtypeStruct((NW, LANES), jnp.float32),
    ),
    scratch_types=[
        pltpu.VMEM((NCH, T), jnp.int32),
        pltpu.VMEM((NCH, T), jnp.int32),
        pltpu.VMEM((VPAD,), jnp.float32),
        pltpu.VMEM((T, PW), jnp.float32),
        pltpu.VMEM((T, PW), jnp.float32),
        pltpu.VMEM((LANES,), jnp.float32),
        pltpu.SemaphoreType.DMA,
    ],
)
def _sc_kernel(tab_hbm, idx_hbm, tgt_hbm, lse_hbm, out_hbm, tail_hbm,
               part_hbm, idx_v, tgt_v, lse_v, buf_a, buf_b, acc_v, sem):
    wid = lax.axis_index("s") * 2 + lax.axis_index("c")
    base = wid * RPW
    pltpu.sync_copy(idx_hbm.at[wid], idx_v)
    pltpu.sync_copy(tgt_hbm.at[wid], tgt_v)
    pltpu.sync_copy(lse_hbm, lse_v)

    def chunk_body(c, acc):
        tok0 = base + c * T
        for p in range(NP):
            buf = buf_a if p % 2 == 0 else buf_b
            pltpu.async_copy(
                tab_hbm.at[p].at[idx_v.at[c]], buf, sem).wait()
            for j in range(T // LANES):
                i_vec = idx_v[c, pl.ds(j * LANES, LANES)]
                t_vec = tgt_v[c, pl.ds(j * LANES, LANES)]
                rid = lax.iota(jnp.int32, LANES) + j * LANES
                t_div = lax.shift_right_logical(t_vec, 7)
                t_mod = lax.bitwise_and(t_vec, 127)
                p_vals = plsc.load_gather(buf, [rid, t_mod])
                if p == 0:
                    acc = acc + plsc.load_gather(lse_v, [i_vec])
                acc = acc - jnp.where(t_div == p, p_vals, 0.0)
            if p == NP - 1:
                pltpu.sync_copy(buf, tail_hbm.at[pl.ds(tok0, T)])
            else:
                pltpu.sync_copy(
                    buf, out_hbm.at[pl.ds(tok0, T), pl.ds(p * PW, PW)])
        return acc

    acc = lax.fori_loop(0, NCH, chunk_body,
                        jnp.zeros((LANES,), jnp.float32))
    acc_v[...] = acc
    pltpu.sync_copy(acc_v, part_hbm.at[wid])


def kernel(params, idx, targets):
    idx3 = idx.astype(jnp.int32).reshape(NW, NCH, T)
    tgt3 = targets.astype(jnp.int32).reshape(NW, NCH, T)
    tab = jnp.pad(params, ((0, 0), (0, VPAD - VOCAB)))
    tab = tab.reshape(VOCAB, NP, PW).transpose(1, 0, 2)
    lse = _row_lse(params)
    logits, tail, partials = _sc_kernel(tab, idx3, tgt3, lse)
    logits = lax.dynamic_update_slice(
        logits, tail[:, :LASTW], (0, (NP - 1) * PW))
    loss = jnp.sum(partials) / BT
    return (loss, logits)


# speedup vs baseline: 2.0839x; 1.3349x over previous
import functools

import jax
import jax.numpy as jnp
from jax import lax
from jax.experimental import pallas as pl
from jax.experimental.pallas import tpu as pltpu
from jax.experimental.pallas import tpu_sc as plsc

VOCAB = 1000
VPAD = 1024
BT = 51200
NW = 32
RPW = BT // NW
CH = 32
NCH = RPW // CH
LANES = 16


def _lse_body(p_ref, o_ref):
    x = p_ref[...]
    m = jnp.max(x, axis=1)
    s = jnp.sum(jnp.exp(x - m[:, None]), axis=1)
    lse = m + jnp.log(s)
    o_ref[...] = jnp.concatenate(
        [lse, jnp.zeros((VPAD - VOCAB,), jnp.float32)])


def _row_lse(params):
    return pl.pallas_call(
        _lse_body,
        out_shape=jax.ShapeDtypeStruct((VPAD,), jnp.float32),
    )(params)


_mesh = plsc.VectorSubcoreMesh(core_axis_name="c", subcore_axis_name="s")


@functools.partial(
    pl.kernel,
    mesh=_mesh,
    compiler_params=pltpu.CompilerParams(needs_layout_passes=False),
    out_type=(
        jax.ShapeDtypeStruct((BT, VPAD), jnp.float32),
        jax.ShapeDtypeStruct((NW, LANES), jnp.float32),
    ),
    scratch_types=[
        pltpu.VMEM((NCH, CH), jnp.int32),
        pltpu.VMEM((NCH, CH), jnp.int32),
        pltpu.VMEM((VPAD,), jnp.float32),
        pltpu.VMEM((CH, VPAD), jnp.float32),
        pltpu.VMEM((CH, VPAD), jnp.float32),
        pltpu.VMEM((LANES,), jnp.float32),
        pltpu.SemaphoreType.DMA,
        pltpu.SemaphoreType.DMA,
    ],
)
def _sc_kernel(tab_hbm, idx_hbm, tgt_hbm, lse_hbm, out_hbm, part_hbm,
               idx_v, tgt_v, lse_v, buf_a, buf_b, acc_v, sem_g, sem_w):
    wid = lax.axis_index("s") * 2 + lax.axis_index("c")
    base = wid * RPW
    pltpu.sync_copy(idx_hbm.at[wid], idx_v)
    pltpu.sync_copy(tgt_hbm.at[wid], tgt_v)
    pltpu.sync_copy(lse_hbm, lse_v)

    def chunk_body(c, acc):
        for s in range(2):
            buf = buf_a if s == 0 else buf_b
            cc = 2 * c + s
            pltpu.async_copy(
                tab_hbm.at[idx_v.at[cc]], buf, sem_g).wait()
            for j in range(CH // LANES):
                i_vec = idx_v[cc, pl.ds(j * LANES, LANES)]
                t_vec = tgt_v[cc, pl.ds(j * LANES, LANES)]
                rid = lax.iota(jnp.int32, LANES) + j * LANES
                p_vals = plsc.load_gather(buf, [rid, t_vec])
                lse_vals = plsc.load_gather(lse_v, [i_vec])
                acc = acc + (lse_vals - p_vals)
            pltpu.sync_copy(
                buf, out_hbm.at[pl.ds(base + cc * CH, CH)])
        return acc

    acc = lax.fori_loop(0, NCH // 2, chunk_body,
                        jnp.zeros((LANES,), jnp.float32))
    acc_v[...] = acc
    pltpu.sync_copy(acc_v, part_hbm.at[wid])


def kernel(params, idx, targets):
    idx3 = idx.astype(jnp.int32).reshape(NW, NCH, CH)
    tgt3 = targets.astype(jnp.int32).reshape(NW, NCH, CH)
    tab = jnp.pad(params, ((0, 0), (0, VPAD - VOCAB)))
    lse = _row_lse(params)
    padded, partials = _sc_kernel(tab, idx3, tgt3, lse)
    logits = padded[:, :VOCAB]
    loss = jnp.sum(partials) / BT
    return (loss, logits)


# speedup vs baseline: 2.2434x; 1.0765x over previous
import functools

import jax
import jax.numpy as jnp
from jax import lax
from jax.experimental import pallas as pl
from jax.experimental.pallas import tpu as pltpu
from jax.experimental.pallas import tpu_sc as plsc

VOCAB = 1000
VPAD = 1024
BT = 51200
NW = 32
RPW = BT // NW
CH = 32
NCH = RPW // CH
LANES = 16


def _lse_body(p_ref, o_ref):
    x = p_ref[...]
    m = jnp.max(x, axis=1)
    s = jnp.sum(jnp.exp(x - m[:, None]), axis=1)
    lse = m + jnp.log(s)
    o_ref[...] = jnp.concatenate(
        [lse, jnp.zeros((VPAD - VOCAB,), jnp.float32)])


def _row_lse(params):
    return pl.pallas_call(
        _lse_body,
        out_shape=jax.ShapeDtypeStruct((VPAD,), jnp.float32),
    )(params)


_mesh = plsc.VectorSubcoreMesh(core_axis_name="c", subcore_axis_name="s")


@functools.partial(
    pl.kernel,
    mesh=_mesh,
    compiler_params=pltpu.CompilerParams(needs_layout_passes=False),
    out_type=(
        jax.ShapeDtypeStruct((BT, VPAD), jnp.float32),
        jax.ShapeDtypeStruct((NW, LANES), jnp.float32),
    ),
    scratch_types=[
        pltpu.VMEM((NCH, CH), jnp.int32),
        pltpu.VMEM((NCH, CH), jnp.int32),
        pltpu.VMEM((VPAD,), jnp.float32),
        pltpu.VMEM((CH, VPAD), jnp.float32),
        pltpu.VMEM((CH, VPAD), jnp.float32),
        pltpu.VMEM((LANES,), jnp.float32),
        pltpu.SemaphoreType.DMA,
        pltpu.SemaphoreType.DMA,
        pltpu.SemaphoreType.DMA,
    ],
)
def _sc_kernel(tab_hbm, idx_hbm, tgt_hbm, lse_hbm, out_hbm, part_hbm,
               idx_v, tgt_v, lse_v, buf_a, buf_b, acc_v,
               sem_g, sem_wa, sem_wb):
    wid = lax.axis_index("s") * 2 + lax.axis_index("c")
    base = wid * RPW
    pltpu.sync_copy(idx_hbm.at[wid], idx_v)
    pltpu.sync_copy(tgt_hbm.at[wid], tgt_v)
    pltpu.sync_copy(lse_hbm, lse_v)

    def chunk_body(c, acc):
        for s in range(2):
            buf = buf_a if s == 0 else buf_b
            sem_w = sem_wa if s == 0 else sem_wb
            cc = 2 * c + s

            @pl.when(cc >= 2)
            def _():
                pltpu.make_async_copy(
                    buf, out_hbm.at[pl.ds(base + (cc - 2) * CH, CH)],
                    sem_w).wait()

            pltpu.async_copy(
                tab_hbm.at[idx_v.at[cc]], buf, sem_g).wait()
            pltpu.async_copy(
                buf, out_hbm.at[pl.ds(base + cc * CH, CH)], sem_w)
            for j in range(CH // LANES):
                i_vec = idx_v[cc, pl.ds(j * LANES, LANES)]
                t_vec = tgt_v[cc, pl.ds(j * LANES, LANES)]
                rid = lax.iota(jnp.int32, LANES) + j * LANES
                p_vals = plsc.load_gather(buf, [rid, t_vec])
                lse_vals = plsc.load_gather(lse_v, [i_vec])
                acc = acc + (lse_vals - p_vals)
        return acc

    acc = lax.fori_loop(0, NCH // 2, chunk_body,
                        jnp.zeros((LANES,), jnp.float32))
    pltpu.make_async_copy(
        buf_a, out_hbm.at[pl.ds(base + (NCH - 2) * CH, CH)], sem_wa).wait()
    pltpu.make_async_copy(
        buf_b, out_hbm.at[pl.ds(base + (NCH - 1) * CH, CH)], sem_wb).wait()
    acc_v[...] = acc
    pltpu.sync_copy(acc_v, part_hbm.at[wid])


def kernel(params, idx, targets):
    idx3 = idx.astype(jnp.int32).reshape(NW, NCH, CH)
    tgt3 = targets.astype(jnp.int32).reshape(NW, NCH, CH)
    tab = jnp.pad(params, ((0, 0), (0, VPAD - VOCAB)))
    lse = _row_lse(params)
    padded, partials = _sc_kernel(tab, idx3, tgt3, lse)
    logits = padded[:, :VOCAB]
    loss = jnp.sum(partials) / BT
    return (loss, logits)


# speedup vs baseline: 2.2443x; 1.0004x over previous
import functools

import jax
import jax.numpy as jnp
from jax import lax
from jax.experimental import pallas as pl
from jax.experimental.pallas import tpu as pltpu
from jax.experimental.pallas import tpu_sc as plsc

VOCAB = 1000
VPAD = 1024
BT = 51200
NW = 32
RPW = BT // NW
CH = 32
NCH = RPW // CH
LANES = 16


def _lse_body(p_ref, o_ref):
    x = p_ref[...]
    m = jnp.max(x, axis=1)
    s = jnp.sum(jnp.exp(x - m[:, None]), axis=1)
    lse = m + jnp.log(s)
    o_ref[...] = jnp.concatenate(
        [lse, jnp.zeros((VPAD - VOCAB,), jnp.float32)])


def _row_lse(params):
    return pl.pallas_call(
        _lse_body,
        out_shape=jax.ShapeDtypeStruct((VPAD,), jnp.float32),
    )(params)


_mesh = plsc.VectorSubcoreMesh(core_axis_name="c", subcore_axis_name="s")


@functools.partial(
    pl.kernel,
    mesh=_mesh,
    compiler_params=pltpu.CompilerParams(needs_layout_passes=False),
    out_type=(
        jax.ShapeDtypeStruct((BT, VPAD), jnp.float32),
        jax.ShapeDtypeStruct((NW, LANES), jnp.float32),
    ),
    scratch_types=[
        pltpu.VMEM((NCH, CH), jnp.int32),
        pltpu.VMEM((NCH, CH), jnp.int32),
        pltpu.VMEM((VPAD,), jnp.float32),
        pltpu.VMEM((CH, VPAD), jnp.float32),
        pltpu.VMEM((CH, VPAD), jnp.float32),
        pltpu.VMEM((CH, VPAD), jnp.float32),
        pltpu.VMEM((LANES,), jnp.float32),
        pltpu.SemaphoreType.DMA,
        pltpu.SemaphoreType.DMA,
        pltpu.SemaphoreType.DMA,
        pltpu.SemaphoreType.DMA,
    ],
)
def _sc_kernel(tab_hbm, idx_hbm, tgt_hbm, lse_hbm, out_hbm, part_hbm,
               idx_v, tgt_v, lse_v, buf_a, buf_b, buf_c, acc_v,
               sem_g, sem_wa, sem_wb, sem_wc):
    wid = lax.axis_index("s") * 2 + lax.axis_index("c")
    base = wid * RPW
    pltpu.sync_copy(idx_hbm.at[wid], idx_v)
    pltpu.sync_copy(tgt_hbm.at[wid], tgt_v)
    pltpu.sync_copy(lse_hbm, lse_v)

    bufs = (buf_a, buf_b, buf_c)
    sems = (sem_wa, sem_wb, sem_wc)

    def sub_step(cc, slot, acc, dyn_guard):
        buf, sem_w = bufs[slot], sems[slot]

        def drain():
            pltpu.make_async_copy(
                buf, out_hbm.at[pl.ds(base + (cc - 3) * CH, CH)],
                sem_w).wait()

        if dyn_guard:
            pl.when(cc >= 3)(drain)
        else:
            drain()

        pltpu.async_copy(tab_hbm.at[idx_v.at[cc]], buf, sem_g).wait()
        pltpu.async_copy(
            buf, out_hbm.at[pl.ds(base + cc * CH, CH)], sem_w)
        for j in range(CH // LANES):
            i_vec = idx_v[cc, pl.ds(j * LANES, LANES)]
            t_vec = tgt_v[cc, pl.ds(j * LANES, LANES)]
            rid = lax.iota(jnp.int32, LANES) + j * LANES
            p_vals = plsc.load_gather(buf, [rid, t_vec])
            lse_vals = plsc.load_gather(lse_v, [i_vec])
            acc = acc + (lse_vals - p_vals)
        return acc

    def chunk_body(c, acc):
        for s in range(3):
            acc = sub_step(3 * c + s, s, acc, dyn_guard=True)
        return acc

    n_main = (NCH // 3) * 3
    acc = lax.fori_loop(0, NCH // 3, chunk_body,
                        jnp.zeros((LANES,), jnp.float32))
    for cc in range(n_main, NCH):
        acc = sub_step(cc, cc % 3, acc, dyn_guard=False)
    for cc in range(NCH - 3, NCH):
        slot = cc % 3
        pltpu.make_async_copy(
            bufs[slot], out_hbm.at[pl.ds(base + cc * CH, CH)],
            sems[slot]).wait()
    acc_v[...] = acc
    pltpu.sync_copy(acc_v, part_hbm.at[wid])


def kernel(params, idx, targets):
    idx3 = idx.astype(jnp.int32).reshape(NW, NCH, CH)
    tgt3 = targets.astype(jnp.int32).reshape(NW, NCH, CH)
    tab = jnp.pad(params, ((0, 0), (0, VPAD - VOCAB)))
    lse = _row_lse(params)
    padded, partials = _sc_kernel(tab, idx3, tgt3, lse)
    logits = padded[:, :VOCAB]
    loss = jnp.sum(partials) / BT
    return (loss, logits)


# speedup vs baseline: 2.2686x; 1.0108x over previous
import functools

import jax
import jax.numpy as jnp
from jax import lax
from jax.experimental import pallas as pl
from jax.experimental.pallas import tpu as pltpu
from jax.experimental.pallas import tpu_sc as plsc

VOCAB = 1000
VPAD = 1024
BT = 51200
NW = 32
RPW = BT // NW
CH = 32
NCH = RPW // CH
LANES = 16


def _lse_body(p_ref, o_ref):
    x = p_ref[...]
    m = jnp.max(x, axis=1)
    s = jnp.sum(jnp.exp(x - m[:, None]), axis=1)
    lse = m + jnp.log(s)
    o_ref[...] = jnp.concatenate(
        [lse, jnp.zeros((VPAD - VOCAB,), jnp.float32)])


def _row_lse(params):
    return pl.pallas_call(
        _lse_body,
        out_shape=jax.ShapeDtypeStruct((VPAD,), jnp.float32),
    )(params)


_mesh = plsc.VectorSubcoreMesh(core_axis_name="c", subcore_axis_name="s")


@functools.partial(
    pl.kernel,
    mesh=_mesh,
    compiler_params=pltpu.CompilerParams(needs_layout_passes=False),
    out_type=(
        jax.ShapeDtypeStruct((BT, VPAD), jnp.float32),
        jax.ShapeDtypeStruct((NW, LANES), jnp.float32),
    ),
    scratch_types=[
        pltpu.VMEM((NCH, CH), jnp.int32),
        pltpu.VMEM((NCH, CH), jnp.int32),
        pltpu.VMEM((VPAD,), jnp.float32),
        pltpu.VMEM((CH, VPAD), jnp.float32),
        pltpu.VMEM((CH, VPAD), jnp.float32),
        pltpu.VMEM((CH, VPAD), jnp.float32),
        pltpu.VMEM((LANES,), jnp.float32),
        pltpu.SemaphoreType.DMA,
        pltpu.SemaphoreType.DMA,
        pltpu.SemaphoreType.DMA,
        pltpu.SemaphoreType.DMA,
        pltpu.SemaphoreType.DMA,
        pltpu.SemaphoreType.DMA,
    ],
)
def _sc_kernel(tab_hbm, idx_hbm, tgt_hbm, lse_hbm, out_hbm, part_hbm,
               idx_v, tgt_v, lse_v, buf_a, buf_b, buf_c, acc_v,
               sem_ga, sem_gb, sem_gc, sem_wa, sem_wb, sem_wc):
    wid = lax.axis_index("s") * 2 + lax.axis_index("c")
    base = wid * RPW
    pltpu.sync_copy(idx_hbm.at[wid], idx_v)
    pltpu.sync_copy(tgt_hbm.at[wid], tgt_v)
    pltpu.sync_copy(lse_hbm, lse_v)

    bufs = (buf_a, buf_b, buf_c)
    gsems = (sem_ga, sem_gb, sem_gc)
    wsems = (sem_wa, sem_wb, sem_wc)

    def start_gather(cc, slot, dyn_guard, static_skip_drain=False):
        buf = bufs[slot]

        def drain():
            pltpu.make_async_copy(
                buf, out_hbm.at[pl.ds(base + (cc - 3) * CH, CH)],
                wsems[slot]).wait()

        if dyn_guard:
            pl.when(cc >= 3)(drain)
        elif not static_skip_drain:
            drain()
        pltpu.async_copy(tab_hbm.at[idx_v.at[cc]], buf, gsems[slot])

    def finish_chunk(cc, slot, acc):
        buf = bufs[slot]
        pltpu.make_async_copy(
            tab_hbm.at[idx_v.at[cc]], buf, gsems[slot]).wait()
        pltpu.async_copy(
            buf, out_hbm.at[pl.ds(base + cc * CH, CH)], wsems[slot])
        for j in range(CH // LANES):
            i_vec = idx_v[cc, pl.ds(j * LANES, LANES)]
            t_vec = tgt_v[cc, pl.ds(j * LANES, LANES)]
            rid = lax.iota(jnp.int32, LANES) + j * LANES
            p_vals = plsc.load_gather(buf, [rid, t_vec])
            lse_vals = plsc.load_gather(lse_v, [i_vec])
            acc = acc + (lse_vals - p_vals)
        return acc

    def chunk_body(c, acc):
        for s in range(3):
            cc = 3 * c + s
            start_gather(cc + 1, (s + 1) % 3, dyn_guard=True)
            acc = finish_chunk(cc, s, acc)
        return acc

    n_main = (NCH // 3) * 3
    start_gather(0, 0, dyn_guard=False, static_skip_drain=True)
    acc = lax.fori_loop(0, NCH // 3, chunk_body,
                        jnp.zeros((LANES,), jnp.float32))
    for cc in range(n_main, NCH):
        if cc + 1 < NCH:
            start_gather(cc + 1, (cc + 1) % 3, dyn_guard=False)
        acc = finish_chunk(cc, cc % 3, acc)
    for cc in range(NCH - 3, NCH):
        slot = cc % 3
        pltpu.make_async_copy(
            bufs[slot], out_hbm.at[pl.ds(base + cc * CH, CH)],
            wsems[slot]).wait()
    acc_v[...] = acc
    pltpu.sync_copy(acc_v, part_hbm.at[wid])


def kernel(params, idx, targets):
    idx3 = idx.astype(jnp.int32).reshape(NW, NCH, CH)
    tgt3 = targets.astype(jnp.int32).reshape(NW, NCH, CH)
    tab = jnp.pad(params, ((0, 0), (0, VPAD - VOCAB)))
    lse = _row_lse(params)
    padded, partials = _sc_kernel(tab, idx3, tgt3, lse)
    logits = padded[:, :VOCAB]
    loss = jnp.sum(partials) / BT
    return (loss, logits)


# speedup vs baseline: 2.2916x; 1.0101x over previous
import functools

import jax
import jax.numpy as jnp
from jax import lax
from jax.experimental import pallas as pl
from jax.experimental.pallas import tpu as pltpu
from jax.experimental.pallas import tpu_sc as plsc

VOCAB = 1000
VPAD = 1024
BT = 51200
NW = 32
RPW = BT // NW
CH = 32
NCH = RPW // CH
LANES = 16


def _lse_body(p_ref, o_ref, t_ref):
    x = p_ref[...]
    m = jnp.max(x, axis=1)
    s = jnp.sum(jnp.exp(x - m[:, None]), axis=1)
    lse = m + jnp.log(s)
    o_ref[...] = jnp.concatenate(
        [lse, jnp.zeros((VPAD - VOCAB,), jnp.float32)])
    t_ref[...] = jnp.pad(x, ((0, 0), (0, VPAD - VOCAB)))


def _row_lse(params):
    return pl.pallas_call(
        _lse_body,
        out_shape=(jax.ShapeDtypeStruct((VPAD,), jnp.float32),
                   jax.ShapeDtypeStruct((VOCAB, VPAD), jnp.float32)),
    )(params)


_mesh = plsc.VectorSubcoreMesh(core_axis_name="c", subcore_axis_name="s")


@functools.partial(
    pl.kernel,
    mesh=_mesh,
    compiler_params=pltpu.CompilerParams(needs_layout_passes=False),
    out_type=(
        jax.ShapeDtypeStruct((BT, VPAD), jnp.float32),
        jax.ShapeDtypeStruct((NW, LANES), jnp.float32),
    ),
    scratch_types=[
        pltpu.VMEM((NCH, CH), jnp.int32),
        pltpu.VMEM((NCH, CH), jnp.int32),
        pltpu.VMEM((VPAD,), jnp.float32),
        pltpu.VMEM((CH, VPAD), jnp.float32),
        pltpu.VMEM((CH, VPAD), jnp.float32),
        pltpu.VMEM((CH, VPAD), jnp.float32),
        pltpu.VMEM((LANES,), jnp.float32),
        pltpu.SemaphoreType.DMA,
        pltpu.SemaphoreType.DMA,
        pltpu.SemaphoreType.DMA,
        pltpu.SemaphoreType.DMA,
        pltpu.SemaphoreType.DMA,
        pltpu.SemaphoreType.DMA,
    ],
)
def _sc_kernel(tab_hbm, idx_hbm, tgt_hbm, lse_hbm, out_hbm, part_hbm,
               idx_v, tgt_v, lse_v, buf_a, buf_b, buf_c, acc_v,
               sem_ga, sem_gb, sem_gc, sem_wa, sem_wb, sem_wc):
    wid = lax.axis_index("s") * 2 + lax.axis_index("c")
    base = wid * RPW
    pltpu.sync_copy(idx_hbm.at[wid], idx_v)
    pltpu.sync_copy(tgt_hbm.at[wid], tgt_v)
    pltpu.sync_copy(lse_hbm, lse_v)

    bufs = (buf_a, buf_b, buf_c)
    gsems = (sem_ga, sem_gb, sem_gc)
    wsems = (sem_wa, sem_wb, sem_wc)

    def start_gather(cc, slot, dyn_guard, static_skip_drain=False):
        buf = bufs[slot]

        def drain():
            pltpu.make_async_copy(
                buf, out_hbm.at[pl.ds(base + (cc - 3) * CH, CH)],
                wsems[slot]).wait()

        if dyn_guard:
            pl.when(cc >= 3)(drain)
        elif not static_skip_drain:
            drain()
        pltpu.async_copy(tab_hbm.at[idx_v.at[cc]], buf, gsems[slot])

    def finish_chunk(cc, slot, acc):
        buf = bufs[slot]
        pltpu.make_async_copy(
            tab_hbm.at[idx_v.at[cc]], buf, gsems[slot]).wait()
        pltpu.async_copy(
            buf, out_hbm.at[pl.ds(base + cc * CH, CH)], wsems[slot])
        for j in range(CH // LANES):
            i_vec = idx_v[cc, pl.ds(j * LANES, LANES)]
            t_vec = tgt_v[cc, pl.ds(j * LANES, LANES)]
            rid = lax.iota(jnp.int32, LANES) + j * LANES
            p_vals = plsc.load_gather(buf, [rid, t_vec])
            lse_vals = plsc.load_gather(lse_v, [i_vec])
            acc = acc + (lse_vals - p_vals)
        return acc

    def chunk_body(c, acc):
        for s in range(3):
            cc = 3 * c + s
            start_gather(cc + 1, (s + 1) % 3, dyn_guard=True)
            acc = finish_chunk(cc, s, acc)
        return acc

    n_main = (NCH // 3) * 3
    start_gather(0, 0, dyn_guard=False, static_skip_drain=True)
    acc = lax.fori_loop(0, NCH // 3, chunk_body,
                        jnp.zeros((LANES,), jnp.float32))
    for cc in range(n_main, NCH):
        if cc + 1 < NCH:
            start_gather(cc + 1, (cc + 1) % 3, dyn_guard=False)
        acc = finish_chunk(cc, cc % 3, acc)
    for cc in range(NCH - 3, NCH):
        slot = cc % 3
        pltpu.make_async_copy(
            bufs[slot], out_hbm.at[pl.ds(base + cc * CH, CH)],
            wsems[slot]).wait()
    acc_v[...] = acc
    pltpu.sync_copy(acc_v, part_hbm.at[wid])


def kernel(params, idx, targets):
    idx3 = idx.astype(jnp.int32).reshape(NW, NCH, CH)
    tgt3 = targets.astype(jnp.int32).reshape(NW, NCH, CH)
    lse, tab = _row_lse(params)
    padded, partials = _sc_kernel(tab, idx3, tgt3, lse)
    logits = padded[:, :VOCAB]
    loss = jnp.sum(partials) / BT
    return (loss, logits)
